# Initial kernel scaffold; baseline (speedup 1.0000x reference)
import jax
import jax.numpy as jnp
from jax.experimental import pallas as pl


def kernel(z, pos, batch, emb_w, i_mlp_w1, i_mlp_b1, i_mlp_w2, i_mlp_b2, i_cf1_w, i_cf2_w, i_cf2_b, i_lin_w, i_lin_b, o1_w, o1_b, o2_w, o2_b):
    raise NotImplementedError("write your pallas kernel here")


# speedup vs baseline: 6.1129x; 6.1129x over previous
import functools

import jax
import jax.numpy as jnp
from jax import lax
from jax.experimental import pallas as pl
from jax.experimental.pallas import tpu as pltpu
from jax.experimental.pallas import tpu_sc as plsc

CUTOFF = 10.0
K = 32
NUM_G = 50
GPAD = 64
H = 128
F = 128
OUT = 256
NI = 3
NGRAPH = 16
N = 8192
E = N * K

LN2 = 0.6931471805599453
INF = jnp.inf
BIGI = 2**30

RB = 128
CW = 256
NT = N // CW

NWORK = 32
NODES_W = N // NWORK
EDGES_W = NODES_W * K
CN = 4
CE = CN * K
NCHUNK = NODES_W // CN


def _ssp(x):
    return jnp.maximum(x, 0.0) + jnp.log(1.0 + jnp.exp(-jnp.abs(x))) - LN2


def _edge_kernel(batch_sm, px_ref, py_ref, pz_ref, brow_ref, post_ref,
                 bcol_ref, nbr_ref, d_ref, vm_ref, scratch_ref):
    blk = pl.program_id(0)
    r0 = blk * RB
    b_lo = batch_sm[r0]
    b_hi = batch_sm[r0 + RB - 1]

    def _bsearch(gt_val, strict):
        def body(_, lohi):
            lo, hi = lohi
            mid = (lo + hi) // 2
            v = batch_sm[mid]
            p = jnp.where(strict, v > gt_val, v >= gt_val)
            return (jnp.where(p, lo, mid + 1), jnp.where(p, mid, hi))
        lo, _ = lax.fori_loop(0, 13, body, (jnp.int32(0), jnp.int32(N)))
        return lo

    s_lo = _bsearch(b_lo, False)
    e_hi = _bsearch(b_hi, True)
    t_lo = s_lo // CW
    t_hi = (e_hi + CW - 1) // CW

    xr = px_ref[...]
    yr = py_ref[...]
    zr = pz_ref[...]
    brow = brow_ref[...]
    rid = r0 + lax.broadcasted_iota(jnp.int32, (RB, 1), 0)

    for t in range(NT):
        @pl.when((t >= t_lo) & (t < t_hi))
        def _fill(t=t):
            xc = post_ref[0:1, t * CW:(t + 1) * CW]
            yc = post_ref[1:2, t * CW:(t + 1) * CW]
            zc = post_ref[2:3, t * CW:(t + 1) * CW]
            bc = bcol_ref[0:1, t * CW:(t + 1) * CW]
            d2 = (xr - xc) ** 2 + (yr - yc) ** 2 + (zr - zc) ** 2
            cid = t * CW + lax.broadcasted_iota(jnp.int32, (RB, CW), 1)
            valid = (brow == bc) & (rid != cid) & (d2 <= CUTOFF * CUTOFF)
            scratch_ref[t] = jnp.where(valid, d2, INF)

    def extract(k, carry):
        nbr_acc, val_acc, prev = carry

        def tscan(t, cur):
            cur_m, cur_i = cur
            cid = t * CW + lax.broadcasted_iota(jnp.int32, (RB, CW), 1)
            tile = jnp.where(cid == prev, INF, scratch_ref[t])
            scratch_ref[t] = tile
            tm = jnp.min(tile, axis=1, keepdims=True)
            ti = jnp.min(jnp.where(tile == tm, cid, BIGI), axis=1,
                         keepdims=True)
            better = tm < cur_m
            tie = tm == cur_m
            new_i = jnp.where(better, ti,
                              jnp.where(tie, jnp.minimum(ti, cur_i), cur_i))
            return jnp.minimum(tm, cur_m), new_i

        rowmin, chosen = lax.fori_loop(
            t_lo, t_hi, tscan,
            (jnp.full((RB, 1), INF, jnp.float32),
             jnp.full((RB, 1), BIGI, jnp.int32)))

        lane = lax.broadcasted_iota(jnp.int32, (RB, K), 1)
        nbr_acc = jnp.where(lane == k, chosen, nbr_acc)
        val_acc = jnp.where(lane == k, rowmin, val_acc)
        return nbr_acc, val_acc, chosen

    nbr_acc, val_acc, _ = lax.fori_loop(
        0, K, extract,
        (jnp.zeros((RB, K), jnp.int32), jnp.full((RB, K), INF, jnp.float32),
         jnp.full((RB, 1), BIGI, jnp.int32)))

    finite = val_acc <= CUTOFF * CUTOFF
    nbr_ref[...] = jnp.where(finite, nbr_acc, 0)
    dval = jnp.sqrt(jnp.where(finite, val_acc, 1.0))
    d_ref[...] = jnp.where(finite, dval, CUTOFF)
    vm_ref[...] = finite.astype(jnp.float32)


def _build_edges(pos, batch_i):
    px = pos[:, 0:1]
    py = pos[:, 1:2]
    pz = pos[:, 2:3]
    post = pos.T
    brow = batch_i[:, None]
    bcol = batch_i[None, :]
    grid_spec = pltpu.PrefetchScalarGridSpec(
        num_scalar_prefetch=1,
        grid=(N // RB,),
        in_specs=[
            pl.BlockSpec((RB, 1), lambda i, sm: (i, 0)),
            pl.BlockSpec((RB, 1), lambda i, sm: (i, 0)),
            pl.BlockSpec((RB, 1), lambda i, sm: (i, 0)),
            pl.BlockSpec((RB, 1), lambda i, sm: (i, 0)),
            pl.BlockSpec((3, N), lambda i, sm: (0, 0)),
            pl.BlockSpec((1, N), lambda i, sm: (0, 0)),
        ],
        out_specs=[
            pl.BlockSpec((RB, K), lambda i, sm: (i, 0)),
            pl.BlockSpec((RB, K), lambda i, sm: (i, 0)),
            pl.BlockSpec((RB, K), lambda i, sm: (i, 0)),
        ],
        scratch_shapes=[pltpu.VMEM((NT, RB, CW), jnp.float32)],
    )
    return pl.pallas_call(
        _edge_kernel,
        grid_spec=grid_spec,
        out_shape=[
            jax.ShapeDtypeStruct((N, K), jnp.int32),
            jax.ShapeDtypeStruct((N, K), jnp.float32),
            jax.ShapeDtypeStruct((N, K), jnp.float32),
        ],
    )(batch_i, px, py, pz, brow, post, bcol)


NB = 512


def _init_kernel(z_ref, emb_ref, cf1_ref, h_ref, hs_ref):
    zb = z_ref[...]
    onehot = (zb == lax.broadcasted_iota(jnp.int32, (NB, 128), 1)
              ).astype(jnp.float32)
    h = jnp.dot(onehot, emb_ref[...], preferred_element_type=jnp.float32)
    h_ref[...] = h
    hs_ref[...] = jnp.dot(h, cf1_ref[...],
                          preferred_element_type=jnp.float32)


def _init_call(z_col, emb_pad, cf1_0):
    return pl.pallas_call(
        _init_kernel,
        grid=(N // NB,),
        in_specs=[
            pl.BlockSpec((NB, 1), lambda i: (i, 0)),
            pl.BlockSpec((128, H), lambda i: (0, 0)),
            pl.BlockSpec((H, F), lambda i: (0, 0)),
        ],
        out_specs=[
            pl.BlockSpec((NB, H), lambda i: (i, 0)),
            pl.BlockSpec((NB, F), lambda i: (i, 0)),
        ],
        out_shape=[
            jax.ShapeDtypeStruct((N, H), jnp.float32),
            jax.ShapeDtypeStruct((N, F), jnp.float32),
        ],
    )(z_col, emb_pad, cf1_0)


EB = 2048
_GSPACE = CUTOFF / (NUM_G - 1)
_GCOEFF = -0.5 / _GSPACE ** 2


def _filter_kernel(d_ref, vm_ref, w1_ref, b1_ref, w2lo_ref, w2hi_ref,
                   b2lo_ref, b2hi_ref, wf0_ref, wf1_ref, wf2_ref):
    d = d_ref[...]
    vm = vm_ref[...]
    g = lax.broadcasted_iota(jnp.int32, (EB, GPAD), 1).astype(
        jnp.float32) * _GSPACE
    ea = jnp.exp(_GCOEFF * (d - g) ** 2).astype(jnp.bfloat16)
    cfac = 0.5 * (jnp.cos(d * (jnp.pi / CUTOFF)) + 1.0) * vm
    outs = (wf0_ref, wf1_ref, wf2_ref)
    for i in range(NI):
        w = jnp.dot(ea, w1_ref[i], preferred_element_type=jnp.float32)
        w = _ssp(w + b1_ref[i:i + 1, :]).astype(jnp.bfloat16)
        a = jnp.dot(w, w2lo_ref[i], preferred_element_type=jnp.float32)
        b = jnp.dot(w, w2hi_ref[i], preferred_element_type=jnp.float32)
        a = (a + b2lo_ref[i:i + 1, :]) * cfac
        b = (b + b2hi_ref[i:i + 1, :]) * cfac
        au = lax.bitcast_convert_type(a.astype(jnp.bfloat16),
                                      jnp.uint16).astype(jnp.int32)
        bu = lax.bitcast_convert_type(b.astype(jnp.bfloat16),
                                      jnp.uint16).astype(jnp.int32)
        outs[i][...] = au | (bu << 16)


def _filter_call(d_e, vm_e, w1p, b1, w2lo, w2hi, b2lo, b2hi):
    return pl.pallas_call(
        _filter_kernel,
        grid=(E // EB,),
        in_specs=[
            pl.BlockSpec((EB, 1), lambda i: (i, 0)),
            pl.BlockSpec((EB, 1), lambda i: (i, 0)),
            pl.BlockSpec((NI, GPAD, F), lambda i: (0, 0, 0)),
            pl.BlockSpec((NI, F), lambda i: (0, 0)),
            pl.BlockSpec((NI, F, F // 2), lambda i: (0, 0, 0)),
            pl.BlockSpec((NI, F, F // 2), lambda i: (0, 0, 0)),
            pl.BlockSpec((NI, F // 2), lambda i: (0, 0)),
            pl.BlockSpec((NI, F // 2), lambda i: (0, 0)),
        ],
        out_specs=[
            pl.BlockSpec((EB, F // 2), lambda i: (i, 0)),
            pl.BlockSpec((EB, F // 2), lambda i: (i, 0)),
            pl.BlockSpec((EB, F // 2), lambda i: (i, 0)),
        ],
        out_shape=[
            jax.ShapeDtypeStruct((E, F // 2), jnp.int32),
            jax.ShapeDtypeStruct((E, F // 2), jnp.int32),
            jax.ShapeDtypeStruct((E, F // 2), jnp.int32),
        ],
    )(d_e, vm_e, w1p, b1, w2lo, w2hi, b2lo, b2hi)


def _sc_agg(hs, wf, nbr_flat):
    mesh = plsc.VectorSubcoreMesh(core_axis_name="c", subcore_axis_name="s")

    @functools.partial(
        pl.kernel,
        out_type=jax.ShapeDtypeStruct((N, H), jnp.float32),
        mesh=mesh,
        scratch_types=[
            pltpu.VMEM((EDGES_W,), jnp.int32),
            pltpu.VMEM((CE, H), jnp.float32),
            pltpu.VMEM((CE, H), jnp.float32),
            pltpu.VMEM((CE, H // 2), jnp.int32),
            pltpu.VMEM((CE, H // 2), jnp.int32),
            pltpu.VMEM((NODES_W, H), jnp.float32),
            pltpu.SemaphoreType.DMA,
            pltpu.SemaphoreType.DMA,
            pltpu.SemaphoreType.DMA,
            pltpu.SemaphoreType.DMA,
        ],
    )
    def body(hs_hbm, wf_hbm, idx_hbm, out_hbm, idx_v, rows0, rows1, wfv0,
             wfv1, agg_v, gs0, gs1, ws0, ws1):
        gsem = (gs0, gs1)
        wsem = (ws0, ws1)
        rows_b = (rows0, rows1)
        wf_b = (wfv0, wfv1)
        wid = lax.axis_index("s") * 2 + lax.axis_index("c")
        ebase = wid * EDGES_W
        pltpu.sync_copy(idx_hbm.at[pl.ds(ebase, EDGES_W)], idx_v)

        def start(c, b):
            pltpu.async_copy(hs_hbm.at[idx_v.at[pl.ds(c * CE, CE)]],
                             rows_b[b], gsem[b])
            pltpu.async_copy(wf_hbm.at[pl.ds(ebase + c * CE, CE), :],
                             wf_b[b], wsem[b])

        def wait(c, b):
            pltpu.make_async_copy(hs_hbm.at[idx_v.at[pl.ds(c * CE, CE)]],
                                  rows_b[b], gsem[b]).wait()
            pltpu.make_async_copy(wf_hbm.at[pl.ds(ebase + c * CE, CE), :],
                                  wf_b[b], wsem[b]).wait()

        start(0, 0)

        def compute(c, b):
            for n in range(CN):
                def ebody(e, acc, n=n):
                    out = list(acc)
                    for g in range(4):
                        w = wf_b[b][n * K + e, pl.ds(g * 16, 16)]
                        wa = lax.bitcast_convert_type(w << 16, jnp.float32)
                        wb = lax.bitcast_convert_type(w & (-65536),
                                                      jnp.float32)
                        r0 = rows_b[b][n * K + e, pl.ds(g * 32, 16)]
                        r1 = rows_b[b][n * K + e, pl.ds(g * 32 + 16, 16)]
                        out[2 * g] = out[2 * g] + r0 * wa
                        out[2 * g + 1] = out[2 * g + 1] + r1 * wb
                    return tuple(out)
                acc = lax.fori_loop(
                    0, K, ebody,
                    tuple(jnp.zeros((16,), jnp.float32) for _ in range(8)))
                for g in range(4):
                    agg_v[c * CN + n, pl.ds(g * 32, 16)] = acc[2 * g]
                    agg_v[c * CN + n, pl.ds(g * 32 + 16, 16)] = acc[2 * g + 1]

        def loop2(cc, carry):
            for b in range(2):
                c = cc * 2 + b

                @pl.when(c + 1 < NCHUNK)
                def _(c=c, b=b):
                    start(c + 1, 1 - b)
                wait(c, b)
                compute(c, b)
            return carry
        lax.fori_loop(0, NCHUNK // 2, loop2, 0)
        pltpu.sync_copy(agg_v, out_hbm.at[pl.ds(wid * NODES_W, NODES_W), :])

    return body(hs, wf, nbr_flat)


def _update_kernel(h_ref, agg_ref, cf2_ref, cf2b_ref, lin_ref, linb_ref,
                   cf1n_ref, hn_ref, hsn_ref):
    m = jnp.dot(agg_ref[...], cf2_ref[...],
                preferred_element_type=jnp.float32) + cf2b_ref[...]
    m = _ssp(m)
    m = jnp.dot(m, lin_ref[...],
                preferred_element_type=jnp.float32) + linb_ref[...]
    hn = h_ref[...] + m
    hn_ref[...] = hn
    hsn_ref[...] = jnp.dot(hn, cf1n_ref[...],
                           preferred_element_type=jnp.float32)


def _update_call(h, agg, cf2, cf2b, linw, linb, cf1n):
    return pl.pallas_call(
        _update_kernel,
        grid=(N // NB,),
        in_specs=[
            pl.BlockSpec((NB, H), lambda i: (i, 0)),
            pl.BlockSpec((NB, H), lambda i: (i, 0)),
            pl.BlockSpec((F, H), lambda i: (0, 0)),
            pl.BlockSpec((1, H), lambda i: (0, 0)),
            pl.BlockSpec((H, H), lambda i: (0, 0)),
            pl.BlockSpec((1, H), lambda i: (0, 0)),
            pl.BlockSpec((H, F), lambda i: (0, 0)),
        ],
        out_specs=[
            pl.BlockSpec((NB, H), lambda i: (i, 0)),
            pl.BlockSpec((NB, F), lambda i: (i, 0)),
        ],
        out_shape=[
            jax.ShapeDtypeStruct((N, H), jnp.float32),
            jax.ShapeDtypeStruct((N, F), jnp.float32),
        ],
    )(h, agg, cf2, cf2b, linw, linb, cf1n)


def _final_kernel(h_ref, agg_ref, cf2_ref, cf2b_ref, lin_ref, linb_ref,
                  o1_ref, o1b_ref, o2_ref, o2b_ref, bat_ref, out_ref):
    m = jnp.dot(agg_ref[...], cf2_ref[...],
                preferred_element_type=jnp.float32) + cf2b_ref[...]
    m = _ssp(m)
    m = jnp.dot(m, lin_ref[...],
                preferred_element_type=jnp.float32) + linb_ref[...]
    hn = h_ref[...] + m
    t = _ssp(jnp.dot(hn, o1_ref[...],
                     preferred_element_type=jnp.float32) + o1b_ref[...])
    y = jnp.dot(t, o2_ref[...],
                preferred_element_type=jnp.float32) + o2b_ref[...]
    onehot = (bat_ref[...] == lax.broadcasted_iota(jnp.int32, (NB, NGRAPH), 1)
              ).astype(jnp.float32)
    part = lax.dot_general(onehot, y, (((0,), (0,)), ((), ())),
                           preferred_element_type=jnp.float32)

    @pl.when(pl.program_id(0) == 0)
    def _():
        out_ref[...] = jnp.zeros_like(out_ref)
    out_ref[...] += part


def _final_call(h, agg, cf2, cf2b, linw, linb, o1w, o1b, o2w, o2b, bat_col):
    return pl.pallas_call(
        _final_kernel,
        grid=(N // NB,),
        in_specs=[
            pl.BlockSpec((NB, H), lambda i: (i, 0)),
            pl.BlockSpec((NB, H), lambda i: (i, 0)),
            pl.BlockSpec((F, H), lambda i: (0, 0)),
            pl.BlockSpec((1, H), lambda i: (0, 0)),
            pl.BlockSpec((H, H), lambda i: (0, 0)),
            pl.BlockSpec((1, H), lambda i: (0, 0)),
            pl.BlockSpec((H, H // 2), lambda i: (0, 0)),
            pl.BlockSpec((1, H // 2), lambda i: (0, 0)),
            pl.BlockSpec((H // 2, OUT), lambda i: (0, 0)),
            pl.BlockSpec((1, OUT), lambda i: (0, 0)),
            pl.BlockSpec((NB, 1), lambda i: (i, 0)),
        ],
        out_specs=pl.BlockSpec((NGRAPH, OUT), lambda i: (0, 0)),
        out_shape=jax.ShapeDtypeStruct((NGRAPH, OUT), jnp.float32),
    )(h, agg, cf2, cf2b, linw, linb, o1w, o1b, o2w, o2b, bat_col)


def kernel(z, pos, batch, emb_w, i_mlp_w1, i_mlp_b1, i_mlp_w2, i_mlp_b2,
           i_cf1_w, i_cf2_w, i_cf2_b, i_lin_w, i_lin_b, o1_w, o1_b, o2_w,
           o2_b):
    batch_i = batch.astype(jnp.int32)
    z_i = z.astype(jnp.int32)

    nbr, dv, vm = _build_edges(pos, batch_i)
    d_e = dv.reshape(E, 1)
    vm_e = vm.reshape(E, 1)
    nbr_flat = nbr.reshape(E)

    lo_cols, hi_cols = [], []
    for g_ in range(4):
        lo_cols += [32 * g_ + i_ for i_ in range(16)]
        hi_cols += [32 * g_ + 16 + i_ for i_ in range(16)]
    lo_cols = jnp.array(lo_cols, jnp.int32)
    hi_cols = jnp.array(hi_cols, jnp.int32)
    w2lo = i_mlp_w2[:, :, lo_cols].astype(jnp.bfloat16)
    w2hi = i_mlp_w2[:, :, hi_cols].astype(jnp.bfloat16)
    b2lo = i_mlp_b2[:, lo_cols]
    b2hi = i_mlp_b2[:, hi_cols]

    emb_pad = jnp.zeros((128, H), jnp.float32).at[:100].set(emb_w)
    h, hs = _init_call(z_i[:, None], emb_pad, i_cf1_w[0])

    w1p = jnp.zeros((NI, GPAD, F), jnp.float32).at[:, :NUM_G].set(
        i_mlp_w1).astype(jnp.bfloat16)
    wfs = _filter_call(d_e, vm_e, w1p, i_mlp_b1, w2lo, w2hi, b2lo, b2hi)

    for i in range(NI):
        agg = _sc_agg(hs, wfs[i], nbr_flat)
        if i < NI - 1:
            h, hs = _update_call(h, agg, i_cf2_w[i], i_cf2_b[i][None, :],
                                 i_lin_w[i], i_lin_b[i][None, :],
                                 i_cf1_w[i + 1])
        else:
            out = _final_call(h, agg, i_cf2_w[i], i_cf2_b[i][None, :],
                              i_lin_w[i], i_lin_b[i][None, :],
                              o1_w, o1_b[None, :], o2_w, o2_b[None, :],
                              batch_i[:, None])
    return out
